# baseline (speedup 1.0000x reference)
import functools

import numpy as np
import jax
import jax.numpy as jnp
from jax import lax
from jax.experimental import pallas as pl
from jax.experimental.pallas import tpu as pltpu
from jax.experimental.pallas import tpu_sc as plsc

D_M = 128
BATCH = 1024
MAXLEN = 200
N_TOK = BATCH * MAXLEN
NC, NS, LANES = 2, 16, 16
NW = NC * NS
PER_W = N_TOK // NW
CHUNK = 40
NCHUNK = PER_W // CHUNK
PHASES = MAXLEN // CHUNK


def _pe_table() -> jnp.ndarray:
    pos = np.arange(MAXLEN, dtype=np.float64)[:, None]
    i = np.arange(D_M)[None, :]
    angle = pos / np.power(10000.0, (2.0 * (i // 2)) / float(D_M))
    pe = np.where(i % 2 == 0, np.sin(angle), np.cos(angle))
    return jnp.asarray(pe, dtype=jnp.float32)


NBUF = 8
LOOKAHEAD = NBUF // 2
ROW_UNROLL = 4

_mesh = plsc.VectorSubcoreMesh(core_axis_name="c", subcore_axis_name="s")


@functools.partial(
    pl.kernel,
    out_type=jax.ShapeDtypeStruct((N_TOK, D_M), jnp.float32),
    mesh=_mesh,
    scratch_types=[
        pltpu.VMEM((PER_W,), jnp.int32),
        pltpu.VMEM((MAXLEN, D_M), jnp.float32),
        pltpu.VMEM((NBUF, CHUNK, D_M), jnp.float32),
    ] + [pltpu.SemaphoreType.DMA] * (2 * NBUF),
)
def _emb_kernel(idx_hbm, pe_hbm, table_hbm, out_hbm, idx_v, pe_v, rows_v,
                *sems):
    sem_g = sems[:NBUF]
    sem_o = sems[NBUF:]
    wid = lax.axis_index("s") * NC + lax.axis_index("c")
    base = wid * PER_W
    pltpu.sync_copy(pe_hbm, pe_v)
    pltpu.sync_copy(idx_hbm.at[pl.ds(base, PER_W)], idx_v)

    def start_gather(t, b):
        pltpu.async_copy(
            table_hbm.at[idx_v.at[pl.ds(t * CHUNK, CHUNK)]],
            rows_v.at[b], sem_g[b])

    def wait_gather(b):
        pltpu.make_async_copy(
            table_hbm.at[idx_v.at[pl.ds(0, CHUNK)]],
            rows_v.at[b], sem_g[b]).wait()

    def start_out(t, b):
        pltpu.async_copy(
            rows_v.at[b], out_hbm.at[pl.ds(base + t * CHUNK, CHUNK)],
            sem_o[b])

    def wait_out(b):
        pltpu.make_async_copy(
            rows_v.at[b], out_hbm.at[pl.ds(base, CHUNK)], sem_o[b]).wait()

    for b in range(LOOKAHEAD):
        start_gather(b, b)

    NGROUP = NCHUNK // NBUF

    def group_body(g, carry):
        for b in range(NBUF):
            u = g * NBUF + b
            bw = (b + LOOKAHEAD) % NBUF
            if b < LOOKAHEAD:
                @pl.when(g >= 1)
                def _():
                    wait_out(bw)

                start_gather(u + LOOKAHEAD, bw)
            else:
                @pl.when(g <= NGROUP - 2)
                def _():
                    wait_out(bw)
                    start_gather(u + LOOKAHEAD, bw)

            wait_gather(b)
            prow = lax.rem(u, PHASES) * CHUNK

            def add_rows(r0, c2):
                for k in range(ROW_UNROLL):
                    r = r0 * ROW_UNROLL + k
                    for j in range(D_M // LANES):
                        sl = pl.ds(j * LANES, LANES)
                        rows_v[b, r, sl] = rows_v[b, r, sl] + pe_v[prow + r, sl]
                return c2

            lax.fori_loop(0, CHUNK // ROW_UNROLL, add_rows, 0)
            start_out(u, b)
        return carry

    lax.fori_loop(0, NGROUP, group_body, 0)
    for b in range(NBUF):
        wait_out(b)


def kernel(inp, emb_table):
    idx = inp.reshape(N_TOK).astype(jnp.int32)
    out = _emb_kernel(idx, _pe_table(), emb_table)
    return out.reshape(BATCH, MAXLEN, D_M)


# speedup vs baseline: 3.0382x; 3.0382x over previous
import functools

import numpy as np
import jax
import jax.numpy as jnp
from jax import lax
from jax.experimental import pallas as pl
from jax.experimental.pallas import tpu as pltpu
from jax.experimental.pallas import tpu_sc as plsc

D_M = 128
BATCH = 1024
MAXLEN = 200
N_TOK = BATCH * MAXLEN
NC, NS, LANES = 2, 16, 16
NW = NC * NS
PER_W = N_TOK // NW
CHUNK = 40
NCHUNK = PER_W // CHUNK
PHASES = MAXLEN // CHUNK


def _pe_table() -> jnp.ndarray:
    pos = np.arange(MAXLEN, dtype=np.float64)[:, None]
    i = np.arange(D_M)[None, :]
    angle = pos / np.power(10000.0, (2.0 * (i // 2)) / float(D_M))
    pe = np.where(i % 2 == 0, np.sin(angle), np.cos(angle))
    return jnp.asarray(pe, dtype=jnp.float32)


NBUF = 8
LOOKAHEAD = NBUF // 2
ROW_UNROLL = 4

_mesh = plsc.VectorSubcoreMesh(core_axis_name="c", subcore_axis_name="s")


@functools.partial(
    pl.kernel,
    out_type=jax.ShapeDtypeStruct((N_TOK, D_M), jnp.float32),
    mesh=_mesh,
    scratch_types=[
        pltpu.VMEM((PER_W,), jnp.int32),
        pltpu.VMEM_SHARED((MAXLEN, D_M), jnp.float32),
        pltpu.VMEM((NBUF, CHUNK, D_M), jnp.float32),
    ] + [pltpu.SemaphoreType.DMA] * (2 * NBUF),
)
def _emb_kernel(idx_hbm, pe_hbm, table_hbm, out_hbm, idx_v, pe_v, rows_v,
                *sems):
    sem_g = sems[:NBUF]
    sem_o = sems[NBUF:]
    sid = lax.axis_index("s")
    wid = sid * NC + lax.axis_index("c")
    base = wid * PER_W

    @pl.when(sid == 0)
    def _():
        pltpu.sync_copy(pe_hbm, pe_v)

    pltpu.sync_copy(idx_hbm.at[pl.ds(base, PER_W)], idx_v)
    plsc.subcore_barrier()

    def start_gather(t, b):
        prow = lax.rem(t, PHASES) * CHUNK
        pltpu.sync_copy(pe_v.at[pl.ds(prow, CHUNK)], rows_v.at[b])
        pltpu.async_copy(
            table_hbm.at[idx_v.at[pl.ds(t * CHUNK, CHUNK)]],
            rows_v.at[b], sem_g[b], add=True)

    def wait_gather(b):
        pltpu.make_async_copy(
            table_hbm.at[idx_v.at[pl.ds(0, CHUNK)]],
            rows_v.at[b], sem_g[b]).wait()

    def start_out(t, b):
        pltpu.async_copy(
            rows_v.at[b], out_hbm.at[pl.ds(base + t * CHUNK, CHUNK)],
            sem_o[b])

    def wait_out(b):
        pltpu.make_async_copy(
            rows_v.at[b], out_hbm.at[pl.ds(base, CHUNK)], sem_o[b]).wait()

    for b in range(LOOKAHEAD):
        start_gather(b, b)

    NGROUP = NCHUNK // NBUF

    def group_body(g, carry):
        for b in range(NBUF):
            u = g * NBUF + b
            bw = (b + LOOKAHEAD) % NBUF
            if b < LOOKAHEAD:
                @pl.when(g >= 1)
                def _():
                    wait_out(bw)

                start_gather(u + LOOKAHEAD, bw)
            else:
                @pl.when(g <= NGROUP - 2)
                def _():
                    wait_out(bw)
                    start_gather(u + LOOKAHEAD, bw)

            wait_gather(b)
            start_out(u, b)
        return carry

    lax.fori_loop(0, NGROUP, group_body, 0)
    for b in range(NBUF):
        wait_out(b)


def kernel(inp, emb_table):
    idx = inp.reshape(N_TOK).astype(jnp.int32)
    out = _emb_kernel(idx, _pe_table(), emb_table)
    return out.reshape(BATCH, MAXLEN, D_M)


# speedup vs baseline: 3.0675x; 1.0097x over previous
import functools

import numpy as np
import jax
import jax.numpy as jnp
from jax import lax
from jax.experimental import pallas as pl
from jax.experimental.pallas import tpu as pltpu
from jax.experimental.pallas import tpu_sc as plsc

D_M = 128
BATCH = 1024
MAXLEN = 200
N_TOK = BATCH * MAXLEN
NC, NS, LANES = 2, 16, 16
NW = NC * NS
PER_W = N_TOK // NW
CHUNK = 80
NCHUNK = PER_W // CHUNK
PE_EXT = MAXLEN + CHUNK


def _pe_table() -> jnp.ndarray:
    pos = (np.arange(PE_EXT, dtype=np.int64) % MAXLEN).astype(np.float64)[:, None]
    i = np.arange(D_M)[None, :]
    angle = pos / np.power(10000.0, (2.0 * (i // 2)) / float(D_M))
    pe = np.where(i % 2 == 0, np.sin(angle), np.cos(angle))
    return jnp.asarray(pe, dtype=jnp.float32)


NBUF = 8
LOOKAHEAD = NBUF // 2
ROW_UNROLL = 4

_mesh = plsc.VectorSubcoreMesh(core_axis_name="c", subcore_axis_name="s")


@functools.partial(
    pl.kernel,
    out_type=jax.ShapeDtypeStruct((N_TOK, D_M), jnp.float32),
    mesh=_mesh,
    scratch_types=[
        pltpu.VMEM((PER_W,), jnp.int32),
        pltpu.VMEM_SHARED((PE_EXT, D_M), jnp.float32),
        pltpu.VMEM((NBUF, CHUNK, D_M), jnp.float32),
    ] + [pltpu.SemaphoreType.DMA] * (2 * NBUF),
)
def _emb_kernel(idx_hbm, pe_hbm, table_hbm, out_hbm, idx_v, pe_v, rows_v,
                *sems):
    sem_g = sems[:NBUF]
    sem_o = sems[NBUF:]
    sid = lax.axis_index("s")
    wid = sid * NC + lax.axis_index("c")
    base = wid * PER_W

    @pl.when(sid == 0)
    def _():
        pltpu.sync_copy(pe_hbm, pe_v)

    pltpu.sync_copy(idx_hbm.at[pl.ds(base, PER_W)], idx_v)
    plsc.subcore_barrier()

    def start_gather(t, b):
        prow = lax.rem(t * CHUNK, MAXLEN)
        pltpu.sync_copy(pe_v.at[pl.ds(prow, CHUNK)], rows_v.at[b])
        pltpu.async_copy(
            table_hbm.at[idx_v.at[pl.ds(t * CHUNK, CHUNK)]],
            rows_v.at[b], sem_g[b], add=True)

    def wait_gather(b):
        pltpu.make_async_copy(
            table_hbm.at[idx_v.at[pl.ds(0, CHUNK)]],
            rows_v.at[b], sem_g[b]).wait()

    def start_out(t, b):
        pltpu.async_copy(
            rows_v.at[b], out_hbm.at[pl.ds(base + t * CHUNK, CHUNK)],
            sem_o[b])

    def wait_out(b):
        pltpu.make_async_copy(
            rows_v.at[b], out_hbm.at[pl.ds(base, CHUNK)], sem_o[b]).wait()

    for b in range(LOOKAHEAD):
        start_gather(b, b)

    NGROUP = NCHUNK // NBUF

    def group_body(g, carry):
        for b in range(NBUF):
            u = g * NBUF + b
            bw = (b + LOOKAHEAD) % NBUF
            if b < LOOKAHEAD:
                @pl.when(g >= 1)
                def _():
                    wait_out(bw)

                start_gather(u + LOOKAHEAD, bw)
            else:
                @pl.when(g <= NGROUP - 2)
                def _():
                    wait_out(bw)
                    start_gather(u + LOOKAHEAD, bw)

            wait_gather(b)
            start_out(u, b)
        return carry

    lax.fori_loop(0, NGROUP, group_body, 0)
    for b in range(NBUF):
        wait_out(b)


def kernel(inp, emb_table):
    idx = inp.reshape(N_TOK).astype(jnp.int32)
    out = _emb_kernel(idx, _pe_table(), emb_table)
    return out.reshape(BATCH, MAXLEN, D_M)
